# baseline (speedup 1.0000x reference)
import jax
import jax.numpy as jnp
from jax.experimental import pallas as pl
from jax.experimental.pallas import tpu as pltpu

_BM = 512
_BN = 1024
_INV_SQRT2 = 0.7071067811865476


def _matmul_gelu_kernel(a_ref, w_ref, b_ref, o_ref, w_bf16):
    si = pl.program_id(1)

    @pl.when(si == 0)
    def _convert():
        w_bf16[...] = w_ref[...].astype(jnp.bfloat16)

    x = jnp.dot(a_ref[...], w_bf16[...],
                preferred_element_type=jnp.float32) + b_ref[...]
    o_ref[...] = x * (0.5 * (1.0 + jax.lax.erf(x * _INV_SQRT2)))


def kernel(hidden_states, W, b):
    batch, seq, d_in = hidden_states.shape
    m = batch * seq
    k_dim, n = W.shape
    a = hidden_states.reshape(m, d_in).astype(jnp.bfloat16)
    b2 = b.reshape(1, n)

    bm, bn = min(_BM, m), min(_BN, n)
    grid = (n // bn, m // bm)

    out = pl.pallas_call(
        _matmul_gelu_kernel,
        grid=grid,
        in_specs=[
            pl.BlockSpec((bm, k_dim), lambda ni, si: (si, 0)),
            pl.BlockSpec((k_dim, bn), lambda ni, si: (0, ni)),
            pl.BlockSpec((1, bn), lambda ni, si: (0, ni)),
        ],
        out_specs=pl.BlockSpec((bm, bn), lambda ni, si: (si, ni)),
        out_shape=jax.ShapeDtypeStruct((m, n), jnp.float32),
        scratch_shapes=[pltpu.VMEM((k_dim, bn), jnp.bfloat16)],
        compiler_params=pltpu.CompilerParams(
            dimension_semantics=("parallel", "arbitrary"),
        ),
    )(a, W, b2)
    return out.reshape(batch, seq, n)
